# baseline (speedup 1.0000x reference)
import jax
import jax.numpy as jnp
from jax.experimental import pallas as pl


def _copy_block(emb_ref, out_ref):
    out_ref[...] = emb_ref[...]


def kernel(x, embedding):
    T = x.shape[1]
    F = embedding.shape[1]
    TILE = 4096
    return pl.pallas_call(
        _copy_block,
        grid=(T // TILE,),
        in_specs=[pl.BlockSpec((TILE, F), lambda i: (i, 0))],
        out_specs=pl.BlockSpec((TILE, F), lambda i: (i, 0)),
        out_shape=jax.ShapeDtypeStruct((T, F), embedding.dtype),
    )(embedding)
